# Initial kernel scaffold; baseline (speedup 1.0000x reference)
import jax
import jax.numpy as jnp
from jax.experimental import pallas as pl


def kernel(user_X, item_X, graph_A, mask_sub_adj, P_symm, W_gcn, W_dense):
    raise NotImplementedError("write your pallas kernel here")


# speedup vs baseline: 241.7782x; 241.7782x over previous
import functools

import jax
import jax.numpy as jnp
from jax.experimental import pallas as pl
from jax.experimental.pallas import tpu as pltpu
from jax.experimental.pallas import tpu_sc as plsc


def _embed_body(x_ref, w_ref, out_ref):
    out_ref[...] = jnp.dot(x_ref[...], w_ref[...],
                           preferred_element_type=jnp.float32)


def _embed(x, w, block=512):
    n, d_in = x.shape
    d_out = w.shape[1]
    return pl.pallas_call(
        _embed_body,
        grid=(n // block,),
        in_specs=[
            pl.BlockSpec((block, d_in), lambda i: (i, 0)),
            pl.BlockSpec((d_in, d_out), lambda i: (0, 0)),
        ],
        out_specs=pl.BlockSpec((block, d_out), lambda i: (i, 0)),
        out_shape=jax.ShapeDtypeStruct((n, d_out), jnp.float32),
    )(x, w)


def _expand_rows_sc(vec_pad, n, w):
    info = plsc.get_sparse_core_info()
    nc, ns = info.num_cores, info.num_subcores
    nw = nc * ns
    rows_per = n // nw
    nbuf = 8
    mesh = plsc.VectorSubcoreMesh(core_axis_name="c", subcore_axis_name="s")

    @functools.partial(
        pl.kernel,
        out_type=jax.ShapeDtypeStruct((n * w,), jnp.float32),
        mesh=mesh,
        scratch_types=([pltpu.VMEM((w,), jnp.float32)] * nbuf
                       + [pltpu.SemaphoreType.DMA] * nbuf
                       + [pltpu.SemaphoreType.DMA] * nbuf),
    )
    def k(vec_hbm, out_hbm, *scratch):
        bufs = scratch[:nbuf]
        sa = scratch[nbuf:2 * nbuf]
        sb = scratch[2 * nbuf:]
        wid = jax.lax.axis_index("s") * nc + jax.lax.axis_index("c")
        base = wid * rows_per

        def mk_in(r, b):
            i = base + r
            off = i * (i + 1) // 2
            a = (off // 8) * 8
            return pltpu.make_async_copy(vec_hbm.at[pl.ds(a, w)],
                                         bufs[b], sa[b])

        def mk_out(r, b):
            i = base + r
            return pltpu.make_async_copy(bufs[b],
                                         out_hbm.at[pl.ds(i * w, w)], sb[b])

        for b in range(nbuf):
            mk_in(b, b).start()

        def group(g, _):
            for b in range(nbuf):
                r = g * nbuf + b
                mk_in(r, b).wait()
                mk_out(r, b).start()
                nxt = r + nbuf

                @pl.when(nxt < rows_per)
                def _(b=b, r=r, nxt=nxt):
                    mk_out(r, b).wait()
                    mk_in(nxt, b).start()

            return 0

        jax.lax.fori_loop(0, rows_per // nbuf, group, 0)

        for b in range(nbuf):
            mk_out(rows_per - nbuf + b, b).wait()

    return k(vec_pad).reshape(n, w)


def _atilde_body(g_ref, m_ref, lmain_ref, lext_ref, at_ref, d_ref):
    bi = pl.program_id(0)
    bj = pl.program_id(1)
    t = g_ref.shape[0]
    mx = jnp.maximum(bi, bj)

    rows_l = mx * t + jax.lax.broadcasted_iota(jnp.int32, (t, 1), 0)
    svec = (rows_l * (rows_l + 1) // 2) % 8
    main = lmain_ref[...]
    ext = lext_ref[...]
    shifted = main
    for s in range(1, 8):
        part = jnp.concatenate([main[:, s:], ext[:, :s]], axis=1)
        shifted = jnp.where(svec == s, part, shifted)

    ssig = jax.nn.sigmoid(shifted)
    rows = bi * t + jax.lax.broadcasted_iota(jnp.int32, (t, t), 0)
    cols = bj * t + jax.lax.broadcasted_iota(jnp.int32, (t, t), 1)
    psig = jnp.where(cols <= rows, ssig, ssig.T)
    f = jnp.where(m_ref[...], psig, jnp.float32(1.0))
    at = g_ref[...] * f
    at_ref[...] = at
    partial = jnp.sum(at, axis=1, keepdims=True)

    @pl.when(bj == 0)
    def _():
        d_ref[...] = partial

    @pl.when(bj != 0)
    def _():
        d_ref[...] += partial


def _atilde(graph, mask, lpad, block=512):
    n = graph.shape[0]
    nb = n // block

    def _lmain(i, j):
        lower = i >= j
        return (jnp.where(lower, i, j), jnp.where(lower, j, i))

    def _lext(i, j):
        lower = i >= j
        mn = jnp.where(lower, j, i)
        return (jnp.where(lower, i, j), (mn + 1) * (block // 128))

    return pl.pallas_call(
        _atilde_body,
        grid=(nb, nb),
        in_specs=[
            pl.BlockSpec((block, block), lambda i, j: (i, j)),
            pl.BlockSpec((block, block), lambda i, j: (i, j)),
            pl.BlockSpec((block, block), _lmain),
            pl.BlockSpec((block, 128), _lext),
        ],
        out_specs=[
            pl.BlockSpec((block, block), lambda i, j: (i, j)),
            pl.BlockSpec((block, 1), lambda i, j: (i, 0)),
        ],
        out_shape=[
            jax.ShapeDtypeStruct((n, n), jnp.float32),
            jax.ShapeDtypeStruct((n, 1), jnp.float32),
        ],
    )(graph, mask, lpad, lpad)


def _spmm_body(at_ref, emb_ref, deg_ref, degr_ref, wd_ref, out_ref):
    d = jax.lax.rsqrt(deg_ref[...] + jnp.float32(1e-7))
    y = emb_ref[...] * d
    acc = jnp.dot(at_ref[...], y, preferred_element_type=jnp.float32)
    d_r = jax.lax.rsqrt(degr_ref[...] + jnp.float32(1e-7))
    h = jnp.maximum(acc * d_r, jnp.float32(0.0))
    out_ref[...] = jnp.dot(h, wd_ref[...].T,
                           preferred_element_type=jnp.float32)


def _spmm_head(at, emb, deg, w_dense, block=256):
    n, d_gcn = emb.shape
    d_out = w_dense.shape[0]
    return pl.pallas_call(
        _spmm_body,
        grid=(n // block,),
        in_specs=[
            pl.BlockSpec((block, n), lambda i: (i, 0)),
            pl.BlockSpec((n, d_gcn), lambda i: (0, 0)),
            pl.BlockSpec((n, 1), lambda i: (0, 0)),
            pl.BlockSpec((block, 1), lambda i: (i, 0)),
            pl.BlockSpec((d_out, d_gcn), lambda i: (0, 0)),
        ],
        out_specs=pl.BlockSpec((block, d_out), lambda i: (i, 0)),
        out_shape=jax.ShapeDtypeStruct((n, d_out), jnp.float32),
    )(at, emb, deg, deg, w_dense)


def kernel(user_X, item_X, graph_A, mask_sub_adj, P_symm, W_gcn, W_dense):
    num_user = user_X.shape[0]
    x = jnp.concatenate([user_X, item_X], axis=0)
    n = x.shape[0]
    w = n + 8
    pad = (n - 1) * n // 2 // 8 * 8 + w - P_symm.shape[0]
    vec_pad = jnp.pad(P_symm, (0, max(pad, 0)))
    emb = _embed(x, W_gcn)
    lpad = _expand_rows_sc(vec_pad, n, w)
    at, deg = _atilde(graph_A, mask_sub_adj, lpad)
    out = _spmm_head(at, emb, deg, W_dense)
    return out[:num_user], out[num_user:]


# speedup vs baseline: 246.1139x; 1.0179x over previous
import functools

import jax
import jax.numpy as jnp
from jax.experimental import pallas as pl
from jax.experimental.pallas import tpu as pltpu
from jax.experimental.pallas import tpu_sc as plsc


def _embed_body(x_ref, w_ref, out_ref):
    out_ref[...] = jnp.dot(x_ref[...], w_ref[...],
                           preferred_element_type=jnp.float32)


def _embed(x, w, block=512):
    n, d_in = x.shape
    d_out = w.shape[1]
    return pl.pallas_call(
        _embed_body,
        grid=(n // block,),
        in_specs=[
            pl.BlockSpec((block, d_in), lambda i: (i, 0)),
            pl.BlockSpec((d_in, d_out), lambda i: (0, 0)),
        ],
        out_specs=pl.BlockSpec((block, d_out), lambda i: (i, 0)),
        out_shape=jax.ShapeDtypeStruct((n, d_out), jnp.float32),
    )(x, w)


def _expand_rows_sc(vec_pad, n, w):
    info = plsc.get_sparse_core_info()
    nc, ns = info.num_cores, info.num_subcores
    nw = nc * ns
    rows_per = n // nw
    nbuf = 8
    mesh = plsc.VectorSubcoreMesh(core_axis_name="c", subcore_axis_name="s")

    @functools.partial(
        pl.kernel,
        out_type=jax.ShapeDtypeStruct((n * w,), jnp.float32),
        mesh=mesh,
        scratch_types=([pltpu.VMEM((w,), jnp.float32)] * nbuf
                       + [pltpu.SemaphoreType.DMA] * nbuf
                       + [pltpu.SemaphoreType.DMA] * nbuf),
    )
    def k(vec_hbm, out_hbm, *scratch):
        bufs = scratch[:nbuf]
        sa = scratch[nbuf:2 * nbuf]
        sb = scratch[2 * nbuf:]
        wid = jax.lax.axis_index("s") * nc + jax.lax.axis_index("c")
        base = wid * rows_per

        def mk_in(r, b):
            i = base + r
            off = i * (i + 1) // 2
            a = (off // 8) * 8
            return pltpu.make_async_copy(vec_hbm.at[pl.ds(a, w)],
                                         bufs[b], sa[b])

        def mk_out(r, b):
            i = base + r
            return pltpu.make_async_copy(bufs[b],
                                         out_hbm.at[pl.ds(i * w, w)], sb[b])

        for b in range(nbuf):
            mk_in(b, b).start()

        def group(g, _):
            for b in range(nbuf):
                r = g * nbuf + b
                mk_in(r, b).wait()
                mk_out(r, b).start()
                nxt = r + nbuf

                @pl.when(nxt < rows_per)
                def _(b=b, r=r, nxt=nxt):
                    mk_out(r, b).wait()
                    mk_in(nxt, b).start()

            return 0

        jax.lax.fori_loop(0, rows_per // nbuf, group, 0)

        for b in range(nbuf):
            mk_out(rows_per - nbuf + b, b).wait()

    return k(vec_pad).reshape(n, w)


def _atilde_body(g_ref, m_ref, lmain_ref, lext_ref, at_ref, d_ref):
    bi = pl.program_id(0)
    bj = pl.program_id(1)
    t = g_ref.shape[0]
    mx = jnp.maximum(bi, bj)

    rows_l = mx * t + jax.lax.broadcasted_iota(jnp.int32, (t, 1), 0)
    svec = (rows_l * (rows_l + 1) // 2) % 8
    main = lmain_ref[...]
    ext = lext_ref[...]
    shifted = main
    for s in range(1, 8):
        part = jnp.concatenate([main[:, s:], ext[:, :s]], axis=1)
        shifted = jnp.where(svec == s, part, shifted)

    ssig = jax.nn.sigmoid(shifted)
    rows = bi * t + jax.lax.broadcasted_iota(jnp.int32, (t, t), 0)
    cols = bj * t + jax.lax.broadcasted_iota(jnp.int32, (t, t), 1)
    psig = jnp.where(cols <= rows, ssig, ssig.T)
    f = jnp.where(m_ref[...], psig, jnp.float32(1.0))
    at = g_ref[...] * f
    at_ref[...] = at.astype(jnp.bfloat16)
    partial = jnp.sum(at, axis=1, keepdims=True)

    @pl.when(bj == 0)
    def _():
        d_ref[...] = partial

    @pl.when(bj != 0)
    def _():
        d_ref[...] += partial


def _atilde(graph, mask, lpad, block=512):
    n = graph.shape[0]
    nb = n // block

    def _lmain(i, j):
        lower = i >= j
        return (jnp.where(lower, i, j), jnp.where(lower, j, i))

    def _lext(i, j):
        lower = i >= j
        mn = jnp.where(lower, j, i)
        return (jnp.where(lower, i, j), (mn + 1) * (block // 128))

    return pl.pallas_call(
        _atilde_body,
        grid=(nb, nb),
        in_specs=[
            pl.BlockSpec((block, block), lambda i, j: (i, j)),
            pl.BlockSpec((block, block), lambda i, j: (i, j)),
            pl.BlockSpec((block, block), _lmain),
            pl.BlockSpec((block, 128), _lext),
        ],
        out_specs=[
            pl.BlockSpec((block, block), lambda i, j: (i, j)),
            pl.BlockSpec((block, 1), lambda i, j: (i, 0)),
        ],
        out_shape=[
            jax.ShapeDtypeStruct((n, n), jnp.bfloat16),
            jax.ShapeDtypeStruct((n, 1), jnp.float32),
        ],
    )(graph, mask, lpad, lpad)


def _spmm_body(at_ref, emb_ref, deg_ref, degr_ref, wd_ref, out_ref):
    d = jax.lax.rsqrt(deg_ref[...] + jnp.float32(1e-7))
    y = emb_ref[...] * d
    acc = jnp.dot(at_ref[...], y, preferred_element_type=jnp.float32)
    d_r = jax.lax.rsqrt(degr_ref[...] + jnp.float32(1e-7))
    h = jnp.maximum(acc * d_r, jnp.float32(0.0))
    out_ref[...] = jnp.dot(h, wd_ref[...].T,
                           preferred_element_type=jnp.float32)


def _spmm_head(at, emb, deg, w_dense, block=256):
    n, d_gcn = emb.shape
    d_out = w_dense.shape[0]
    return pl.pallas_call(
        _spmm_body,
        grid=(n // block,),
        in_specs=[
            pl.BlockSpec((block, n), lambda i: (i, 0)),
            pl.BlockSpec((n, d_gcn), lambda i: (0, 0)),
            pl.BlockSpec((n, 1), lambda i: (0, 0)),
            pl.BlockSpec((block, 1), lambda i: (i, 0)),
            pl.BlockSpec((d_out, d_gcn), lambda i: (0, 0)),
        ],
        out_specs=pl.BlockSpec((block, d_out), lambda i: (i, 0)),
        out_shape=jax.ShapeDtypeStruct((n, d_out), jnp.float32),
    )(at, emb, deg, deg, w_dense)


def kernel(user_X, item_X, graph_A, mask_sub_adj, P_symm, W_gcn, W_dense):
    num_user = user_X.shape[0]
    x = jnp.concatenate([user_X, item_X], axis=0)
    n = x.shape[0]
    w = n + 8
    pad = (n - 1) * n // 2 // 8 * 8 + w - P_symm.shape[0]
    vec_pad = jnp.pad(P_symm, (0, max(pad, 0)))
    emb = _embed(x, W_gcn)
    lpad = _expand_rows_sc(vec_pad, n, w)
    at, deg = _atilde(graph_A, mask_sub_adj, lpad)
    out = _spmm_head(at, emb, deg, W_dense)
    return out[:num_user], out[num_user:]


# speedup vs baseline: 306.9652x; 1.2472x over previous
import functools

import jax
import jax.numpy as jnp
from jax.experimental import pallas as pl
from jax.experimental.pallas import tpu as pltpu
from jax.experimental.pallas import tpu_sc as plsc


def _embed_body(x_ref, w_ref, out_ref):
    out_ref[...] = jnp.dot(x_ref[...], w_ref[...],
                           preferred_element_type=jnp.float32)


def _embed(x, w, block=512):
    n, d_in = x.shape
    d_out = w.shape[1]
    return pl.pallas_call(
        _embed_body,
        grid=(n // block,),
        in_specs=[
            pl.BlockSpec((block, d_in), lambda i: (i, 0)),
            pl.BlockSpec((d_in, d_out), lambda i: (0, 0)),
        ],
        out_specs=pl.BlockSpec((block, d_out), lambda i: (i, 0)),
        out_shape=jax.ShapeDtypeStruct((n, d_out), jnp.float32),
    )(x, w)


def _expand_rows_sc(vec_pad, n, w):
    info = plsc.get_sparse_core_info()
    nc, ns = info.num_cores, info.num_subcores
    nw = nc * ns
    rows_per = n // nw
    nbuf = 8
    mesh = plsc.VectorSubcoreMesh(core_axis_name="c", subcore_axis_name="s")

    @functools.partial(
        pl.kernel,
        out_type=jax.ShapeDtypeStruct((n * w,), jnp.float32),
        mesh=mesh,
        scratch_types=([pltpu.VMEM((w,), jnp.float32)] * nbuf
                       + [pltpu.SemaphoreType.DMA] * nbuf
                       + [pltpu.SemaphoreType.DMA] * nbuf),
    )
    def k(vec_hbm, out_hbm, *scratch):
        bufs = scratch[:nbuf]
        sa = scratch[nbuf:2 * nbuf]
        sb = scratch[2 * nbuf:]
        wid = jax.lax.axis_index("s") * nc + jax.lax.axis_index("c")
        base = wid * rows_per

        def mk_in(r, b):
            i = base + r
            off = i * (i + 1) // 2
            a = (off // 8) * 8
            return pltpu.make_async_copy(vec_hbm.at[pl.ds(a, w)],
                                         bufs[b], sa[b])

        def mk_out(r, b):
            i = base + r
            return pltpu.make_async_copy(bufs[b],
                                         out_hbm.at[pl.ds(i * w, w)], sb[b])

        for b in range(nbuf):
            mk_in(b, b).start()

        def group(g, _):
            for b in range(nbuf):
                r = g * nbuf + b
                mk_in(r, b).wait()
                mk_out(r, b).start()
                nxt = r + nbuf

                @pl.when(nxt < rows_per)
                def _(b=b, r=r, nxt=nxt):
                    mk_out(r, b).wait()
                    mk_in(nxt, b).start()

            return 0

        jax.lax.fori_loop(0, rows_per // nbuf, group, 0)

        for b in range(nbuf):
            mk_out(rows_per - nbuf + b, b).wait()

    return k(vec_pad).reshape(n, w)


def _atilde_body(g_ref, m_ref, lmain_ref, lext_ref, at_ref, d_ref):
    bi = pl.program_id(0)
    bj = pl.program_id(1)
    t = g_ref.shape[0]
    mx = jnp.maximum(bi, bj)

    rows_l = mx * t + jax.lax.broadcasted_iota(jnp.int32, (t, 1), 0)
    svec = (rows_l * (rows_l + 1) // 2) % 8
    allp = jnp.concatenate([lmain_ref[...], lext_ref[:, :8]], axis=1)
    x4 = jnp.where((svec & 4) != 0, allp[:, 4:t + 8], allp[:, 0:t + 4])
    x2 = jnp.where((svec & 2) != 0, x4[:, 2:t + 4], x4[:, 0:t + 2])
    shifted = jnp.where((svec & 1) != 0, x2[:, 1:t + 1], x2[:, 0:t])

    ssig = jax.nn.sigmoid(shifted)
    g = g_ref[...]
    m = m_ref[...]

    @pl.when(bi > bj)
    def _():
        at_ref[...] = (g * jnp.where(m, ssig, jnp.float32(1.0))
                       ).astype(jnp.bfloat16)

    @pl.when(bi < bj)
    def _():
        at_ref[...] = (g * jnp.where(m, ssig.T, jnp.float32(1.0))
                       ).astype(jnp.bfloat16)

    @pl.when(bi == bj)
    def _():
        lo = (jax.lax.broadcasted_iota(jnp.int32, (t, t), 1)
              <= jax.lax.broadcasted_iota(jnp.int32, (t, t), 0))
        psig = jnp.where(lo, ssig, ssig.T)
        at_ref[...] = (g * jnp.where(m, psig, jnp.float32(1.0))
                       ).astype(jnp.bfloat16)

    partial = jnp.sum(at_ref[...].astype(jnp.float32), axis=1, keepdims=True)

    @pl.when(bj == 0)
    def _():
        d_ref[...] = partial

    @pl.when(bj != 0)
    def _():
        d_ref[...] += partial


def _atilde(graph, mask, lpad, block=512):
    n = graph.shape[0]
    nb = n // block

    def _lmain(i, j):
        lower = i >= j
        return (jnp.where(lower, i, j), jnp.where(lower, j, i))

    def _lext(i, j):
        lower = i >= j
        mn = jnp.where(lower, j, i)
        return (jnp.where(lower, i, j), (mn + 1) * (block // 128))

    return pl.pallas_call(
        _atilde_body,
        grid=(nb, nb),
        in_specs=[
            pl.BlockSpec((block, block), lambda i, j: (i, j)),
            pl.BlockSpec((block, block), lambda i, j: (i, j)),
            pl.BlockSpec((block, block), _lmain),
            pl.BlockSpec((block, 128), _lext),
        ],
        out_specs=[
            pl.BlockSpec((block, block), lambda i, j: (i, j)),
            pl.BlockSpec((block, 1), lambda i, j: (i, 0)),
        ],
        out_shape=[
            jax.ShapeDtypeStruct((n, n), jnp.bfloat16),
            jax.ShapeDtypeStruct((n, 1), jnp.float32),
        ],
    )(graph, mask, lpad, lpad)


def _spmm_body(at_ref, emb_ref, deg_ref, degr_ref, wd_ref, out_ref):
    d = jax.lax.rsqrt(deg_ref[...] + jnp.float32(1e-7))
    y = emb_ref[...] * d
    acc = jnp.dot(at_ref[...], y, preferred_element_type=jnp.float32)
    d_r = jax.lax.rsqrt(degr_ref[...] + jnp.float32(1e-7))
    h = jnp.maximum(acc * d_r, jnp.float32(0.0))
    out_ref[...] = jnp.dot(h, wd_ref[...].T,
                           preferred_element_type=jnp.float32)


def _spmm_head(at, emb, deg, w_dense, block=256):
    n, d_gcn = emb.shape
    d_out = w_dense.shape[0]
    return pl.pallas_call(
        _spmm_body,
        grid=(n // block,),
        in_specs=[
            pl.BlockSpec((block, n), lambda i: (i, 0)),
            pl.BlockSpec((n, d_gcn), lambda i: (0, 0)),
            pl.BlockSpec((n, 1), lambda i: (0, 0)),
            pl.BlockSpec((block, 1), lambda i: (i, 0)),
            pl.BlockSpec((d_out, d_gcn), lambda i: (0, 0)),
        ],
        out_specs=pl.BlockSpec((block, d_out), lambda i: (i, 0)),
        out_shape=jax.ShapeDtypeStruct((n, d_out), jnp.float32),
    )(at, emb, deg, deg, w_dense)


def kernel(user_X, item_X, graph_A, mask_sub_adj, P_symm, W_gcn, W_dense):
    num_user = user_X.shape[0]
    x = jnp.concatenate([user_X, item_X], axis=0)
    n = x.shape[0]
    w = n + 8
    pad = (n - 1) * n // 2 // 8 * 8 + w - P_symm.shape[0]
    vec_pad = jnp.pad(P_symm, (0, max(pad, 0)))
    emb = _embed(x, W_gcn)
    lpad = _expand_rows_sc(vec_pad, n, w)
    at, deg = _atilde(graph_A, mask_sub_adj, lpad)
    out = _spmm_head(at, emb, deg, W_dense)
    return out[:num_user], out[num_user:]


# speedup vs baseline: 356.4605x; 1.1612x over previous
import functools

import jax
import jax.numpy as jnp
from jax.experimental import pallas as pl
from jax.experimental.pallas import tpu as pltpu
from jax.experimental.pallas import tpu_sc as plsc


def _embed_body(x_ref, w_ref, out_ref):
    out_ref[...] = jnp.dot(x_ref[...], w_ref[...],
                           preferred_element_type=jnp.float32)


def _embed(x, w, block=512):
    n, d_in = x.shape
    d_out = w.shape[1]
    return pl.pallas_call(
        _embed_body,
        grid=(n // block,),
        in_specs=[
            pl.BlockSpec((block, d_in), lambda i: (i, 0)),
            pl.BlockSpec((d_in, d_out), lambda i: (0, 0)),
        ],
        out_specs=pl.BlockSpec((block, d_out), lambda i: (i, 0)),
        out_shape=jax.ShapeDtypeStruct((n, d_out), jnp.float32),
    )(x, w)


def _expand_rows_sc(vec_pad, n, w):
    info = plsc.get_sparse_core_info()
    nc, ns = info.num_cores, info.num_subcores
    nw = nc * ns
    rows_per = n // nw
    nbuf = 8
    mesh = plsc.VectorSubcoreMesh(core_axis_name="c", subcore_axis_name="s")

    @functools.partial(
        pl.kernel,
        out_type=jax.ShapeDtypeStruct((n * w,), jnp.float32),
        mesh=mesh,
        scratch_types=([pltpu.VMEM((w,), jnp.float32)] * nbuf
                       + [pltpu.SemaphoreType.DMA] * nbuf
                       + [pltpu.SemaphoreType.DMA] * nbuf),
    )
    def k(vec_hbm, out_hbm, *scratch):
        bufs = scratch[:nbuf]
        sa = scratch[nbuf:2 * nbuf]
        sb = scratch[2 * nbuf:]
        wid = jax.lax.axis_index("s") * nc + jax.lax.axis_index("c")
        base = wid * rows_per

        def _in(r, b, do_wait):
            i = base + r
            off = i * (i + 1) // 2
            a = (off // 8) * 8

            @pl.when(i != n - 1)
            def _():
                c = pltpu.make_async_copy(vec_hbm.at[pl.ds(a, w)],
                                          bufs[b], sa[b])
                c.wait() if do_wait else c.start()

            @pl.when(i == n - 1)
            def _():
                c = pltpu.make_async_copy(vec_hbm.at[pl.ds(a, w - 8)],
                                          bufs[b].at[pl.ds(0, w - 8)], sa[b])
                c.wait() if do_wait else c.start()

        def mk_out(r, b):
            i = base + r
            return pltpu.make_async_copy(bufs[b],
                                         out_hbm.at[pl.ds(i * w, w)], sb[b])

        for b in range(nbuf):
            _in(b, b, False)

        def group(g, _):
            for b in range(nbuf):
                r = g * nbuf + b
                _in(r, b, True)
                mk_out(r, b).start()
                nxt = r + nbuf

                @pl.when(nxt < rows_per)
                def _(b=b, r=r, nxt=nxt):
                    mk_out(r, b).wait()
                    _in(nxt, b, False)

            return 0

        jax.lax.fori_loop(0, rows_per // nbuf, group, 0)

        for b in range(nbuf):
            mk_out(rows_per - nbuf + b, b).wait()

    return k(vec_pad).reshape(n, w)


def _atilde_body(g_ref, m_ref, lmain_ref, lext_ref, at_ref, d_ref):
    bi = pl.program_id(0)
    bj = pl.program_id(1)
    t = g_ref.shape[0]
    mx = jnp.maximum(bi, bj)

    rows_l = mx * t + jax.lax.broadcasted_iota(jnp.int32, (t, 1), 0)
    svec = (rows_l * (rows_l + 1) // 2) % 8
    allp = jnp.concatenate([lmain_ref[...], lext_ref[:, :8]], axis=1)
    x4 = jnp.where((svec & 4) != 0, allp[:, 4:t + 8], allp[:, 0:t + 4])
    x2 = jnp.where((svec & 2) != 0, x4[:, 2:t + 4], x4[:, 0:t + 2])
    shifted = jnp.where((svec & 1) != 0, x2[:, 1:t + 1], x2[:, 0:t])

    ssig = jax.nn.sigmoid(shifted)
    g = g_ref[...]
    m = m_ref[...]

    @pl.when(bi > bj)
    def _():
        at_ref[...] = (g * jnp.where(m, ssig, jnp.float32(1.0))
                       ).astype(jnp.bfloat16)

    @pl.when(bi < bj)
    def _():
        at_ref[...] = (g * jnp.where(m, ssig.T, jnp.float32(1.0))
                       ).astype(jnp.bfloat16)

    @pl.when(bi == bj)
    def _():
        lo = (jax.lax.broadcasted_iota(jnp.int32, (t, t), 1)
              <= jax.lax.broadcasted_iota(jnp.int32, (t, t), 0))
        psig = jnp.where(lo, ssig, ssig.T)
        at_ref[...] = (g * jnp.where(m, psig, jnp.float32(1.0))
                       ).astype(jnp.bfloat16)

    partial = jnp.sum(at_ref[...].astype(jnp.float32), axis=1, keepdims=True)

    @pl.when(bj == 0)
    def _():
        d_ref[...] = partial

    @pl.when(bj != 0)
    def _():
        d_ref[...] += partial


def _atilde(graph, mask, lpad, block=None):
    n = graph.shape[0]
    if block is None:
        block = next(b for b in (768, 512, 256, 128) if n % b == 0)
    nb = n // block

    def _lmain(i, j):
        lower = i >= j
        return (jnp.where(lower, i, j), jnp.where(lower, j, i))

    def _lext(i, j):
        lower = i >= j
        mn = jnp.where(lower, j, i)
        return (jnp.where(lower, i, j), (mn + 1) * (block // 128))

    return pl.pallas_call(
        _atilde_body,
        grid=(nb, nb),
        in_specs=[
            pl.BlockSpec((block, block), lambda i, j: (i, j)),
            pl.BlockSpec((block, block), lambda i, j: (i, j)),
            pl.BlockSpec((block, block), _lmain),
            pl.BlockSpec((block, 128), _lext),
        ],
        out_specs=[
            pl.BlockSpec((block, block), lambda i, j: (i, j)),
            pl.BlockSpec((block, 1), lambda i, j: (i, 0)),
        ],
        out_shape=[
            jax.ShapeDtypeStruct((n, n), jnp.bfloat16),
            jax.ShapeDtypeStruct((n, 1), jnp.float32),
        ],
    )(graph, mask, lpad, lpad)


def _spmm_body(at_ref, emb_ref, deg_ref, degr_ref, wd_ref, out_ref, y_ref):
    @pl.when(pl.program_id(0) == 0)
    def _():
        d = jax.lax.rsqrt(deg_ref[...] + jnp.float32(1e-7))
        y_ref[...] = emb_ref[...] * d

    acc = jnp.dot(at_ref[...], y_ref[...],
                  preferred_element_type=jnp.float32)
    d_r = jax.lax.rsqrt(degr_ref[...] + jnp.float32(1e-7))
    h = jnp.maximum(acc * d_r, jnp.float32(0.0))
    out_ref[...] = jnp.dot(h, wd_ref[...].T,
                           preferred_element_type=jnp.float32)


def _spmm_head(at, emb, deg, w_dense, block=256):
    n, d_gcn = emb.shape
    d_out = w_dense.shape[0]
    return pl.pallas_call(
        _spmm_body,
        grid=(n // block,),
        in_specs=[
            pl.BlockSpec((block, n), lambda i: (i, 0)),
            pl.BlockSpec((n, d_gcn), lambda i: (0, 0)),
            pl.BlockSpec((n, 1), lambda i: (0, 0)),
            pl.BlockSpec((block, 1), lambda i: (i, 0)),
            pl.BlockSpec((d_out, d_gcn), lambda i: (0, 0)),
        ],
        out_specs=pl.BlockSpec((block, d_out), lambda i: (i, 0)),
        out_shape=jax.ShapeDtypeStruct((n, d_out), jnp.float32),
        scratch_shapes=[pltpu.VMEM((n, d_gcn), jnp.float32)],
    )(at, emb, deg, deg, w_dense)


def kernel(user_X, item_X, graph_A, mask_sub_adj, P_symm, W_gcn, W_dense):
    num_user = user_X.shape[0]
    x = jnp.concatenate([user_X, item_X], axis=0)
    n = x.shape[0]
    w = n + 8
    emb = _embed(x, W_gcn)
    lpad = _expand_rows_sc(P_symm, n, w)
    at, deg = _atilde(graph_A, mask_sub_adj, lpad)
    out = _spmm_head(at, emb, deg, W_dense)
    return out[:num_user], out[num_user:]


# speedup vs baseline: 366.0559x; 1.0269x over previous
import functools

import jax
import jax.numpy as jnp
from jax.experimental import pallas as pl
from jax.experimental.pallas import tpu as pltpu
from jax.experimental.pallas import tpu_sc as plsc


def _embed_body(x_ref, w_ref, out_ref):
    out_ref[...] = jnp.dot(x_ref[...], w_ref[...],
                           preferred_element_type=jnp.float32)


def _embed(x, w, block=512):
    n, d_in = x.shape
    d_out = w.shape[1]
    return pl.pallas_call(
        _embed_body,
        grid=(n // block,),
        in_specs=[
            pl.BlockSpec((block, d_in), lambda i: (i, 0)),
            pl.BlockSpec((d_in, d_out), lambda i: (0, 0)),
        ],
        out_specs=pl.BlockSpec((block, d_out), lambda i: (i, 0)),
        out_shape=jax.ShapeDtypeStruct((n, d_out), jnp.float32),
    )(x, w)


def _expand_rows_sc(vec_pad, n, w):
    info = plsc.get_sparse_core_info()
    nc, ns = info.num_cores, info.num_subcores
    nw = nc * ns
    rows_per = n // nw
    nbuf = 8
    mesh = plsc.VectorSubcoreMesh(core_axis_name="c", subcore_axis_name="s")

    @functools.partial(
        pl.kernel,
        out_type=jax.ShapeDtypeStruct((n * w,), jnp.float32),
        mesh=mesh,
        scratch_types=([pltpu.VMEM((w,), jnp.float32)] * nbuf
                       + [pltpu.SemaphoreType.DMA] * nbuf
                       + [pltpu.SemaphoreType.DMA] * nbuf),
    )
    def k(vec_hbm, out_hbm, *scratch):
        bufs = scratch[:nbuf]
        sa = scratch[nbuf:2 * nbuf]
        sb = scratch[2 * nbuf:]
        wid = jax.lax.axis_index("s") * nc + jax.lax.axis_index("c")
        base = wid * rows_per

        def _in(r, b, do_wait):
            i = base + r
            off = i * (i + 1) // 2
            a = (off // 8) * 8

            @pl.when(i != n - 1)
            def _():
                c = pltpu.make_async_copy(vec_hbm.at[pl.ds(a, w)],
                                          bufs[b], sa[b])
                c.wait() if do_wait else c.start()

            @pl.when(i == n - 1)
            def _():
                c = pltpu.make_async_copy(vec_hbm.at[pl.ds(a, w - 8)],
                                          bufs[b].at[pl.ds(0, w - 8)], sa[b])
                c.wait() if do_wait else c.start()

        def mk_out(r, b):
            i = base + r
            return pltpu.make_async_copy(bufs[b],
                                         out_hbm.at[pl.ds(i * w, w)], sb[b])

        for b in range(nbuf):
            _in(b, b, False)

        def group(g, _):
            for b in range(nbuf):
                r = g * nbuf + b
                _in(r, b, True)
                mk_out(r, b).start()
                nxt = r + nbuf

                @pl.when(nxt < rows_per)
                def _(b=b, r=r, nxt=nxt):
                    mk_out(r, b).wait()
                    _in(nxt, b, False)

            return 0

        jax.lax.fori_loop(0, rows_per // nbuf, group, 0)

        for b in range(nbuf):
            mk_out(rows_per - nbuf + b, b).wait()

    return k(vec_pad).reshape(n, w)


def _atilde_body(g_ref, m_ref, lmain_ref, lext_ref, at_ref, d_ref):
    bi = pl.program_id(0)
    bj = pl.program_id(1)
    t = g_ref.shape[0]
    mx = jnp.maximum(bi, bj)

    rows_l = mx * t + jax.lax.broadcasted_iota(jnp.int32, (t, 1), 0)
    svec = (rows_l * (rows_l + 1) // 2) % 8
    allp = jnp.concatenate([lmain_ref[...], lext_ref[:, :8]], axis=1)
    x4 = jnp.where((svec & 4) != 0, allp[:, 4:t + 8], allp[:, 0:t + 4])
    x2 = jnp.where((svec & 2) != 0, x4[:, 2:t + 4], x4[:, 0:t + 2])
    shifted = jnp.where((svec & 1) != 0, x2[:, 1:t + 1], x2[:, 0:t])

    ssig = jax.nn.sigmoid(shifted)
    g = g_ref[...]
    m = m_ref[...]

    @pl.when(bi > bj)
    def _():
        at_ref[...] = (g * jnp.where(m, ssig, jnp.float32(1.0))
                       ).astype(jnp.bfloat16)

    @pl.when(bi < bj)
    def _():
        at_ref[...] = (g * jnp.where(m, ssig.T, jnp.float32(1.0))
                       ).astype(jnp.bfloat16)

    @pl.when(bi == bj)
    def _():
        lo = (jax.lax.broadcasted_iota(jnp.int32, (t, t), 1)
              <= jax.lax.broadcasted_iota(jnp.int32, (t, t), 0))
        psig = jnp.where(lo, ssig, ssig.T)
        at_ref[...] = (g * jnp.where(m, psig, jnp.float32(1.0))
                       ).astype(jnp.bfloat16)

    partial = jnp.sum(at_ref[...].astype(jnp.float32), axis=1, keepdims=True)

    @pl.when(bj == 0)
    def _():
        d_ref[...] = partial

    @pl.when(bj != 0)
    def _():
        d_ref[...] += partial


def _atilde(graph, mask, lpad, block=None):
    n = graph.shape[0]
    if block is None:
        block = next(b for b in (1024, 768, 512, 256, 128) if n % b == 0)
    nb = n // block

    def _lmain(i, j):
        lower = i >= j
        return (jnp.where(lower, i, j), jnp.where(lower, j, i))

    def _lext(i, j):
        lower = i >= j
        mn = jnp.where(lower, j, i)
        return (jnp.where(lower, i, j), (mn + 1) * (block // 128))

    return pl.pallas_call(
        _atilde_body,
        grid=(nb, nb),
        in_specs=[
            pl.BlockSpec((block, block), lambda i, j: (i, j)),
            pl.BlockSpec((block, block), lambda i, j: (i, j)),
            pl.BlockSpec((block, block), _lmain),
            pl.BlockSpec((block, 128), _lext),
        ],
        out_specs=[
            pl.BlockSpec((block, block), lambda i, j: (i, j)),
            pl.BlockSpec((block, 1), lambda i, j: (i, 0)),
        ],
        out_shape=[
            jax.ShapeDtypeStruct((n, n), jnp.bfloat16),
            jax.ShapeDtypeStruct((n, 1), jnp.float32),
        ],
    )(graph, mask, lpad, lpad)


def _spmm_body(at_ref, emb_ref, deg_ref, degr_ref, wd_ref, out_ref, y_ref):
    @pl.when(pl.program_id(0) == 0)
    def _():
        d = jax.lax.rsqrt(deg_ref[...] + jnp.float32(1e-7))
        y_ref[...] = emb_ref[...] * d

    acc = jnp.dot(at_ref[...], y_ref[...],
                  preferred_element_type=jnp.float32)
    d_r = jax.lax.rsqrt(degr_ref[...] + jnp.float32(1e-7))
    h = jnp.maximum(acc * d_r, jnp.float32(0.0))
    out_ref[...] = jnp.dot(h, wd_ref[...].T,
                           preferred_element_type=jnp.float32)


def _spmm_head(at, emb, deg, w_dense, block=256):
    n, d_gcn = emb.shape
    d_out = w_dense.shape[0]
    return pl.pallas_call(
        _spmm_body,
        grid=(n // block,),
        in_specs=[
            pl.BlockSpec((block, n), lambda i: (i, 0)),
            pl.BlockSpec((n, d_gcn), lambda i: (0, 0)),
            pl.BlockSpec((n, 1), lambda i: (0, 0)),
            pl.BlockSpec((block, 1), lambda i: (i, 0)),
            pl.BlockSpec((d_out, d_gcn), lambda i: (0, 0)),
        ],
        out_specs=pl.BlockSpec((block, d_out), lambda i: (i, 0)),
        out_shape=jax.ShapeDtypeStruct((n, d_out), jnp.float32),
        scratch_shapes=[pltpu.VMEM((n, d_gcn), jnp.float32)],
    )(at, emb, deg, deg, w_dense)


def kernel(user_X, item_X, graph_A, mask_sub_adj, P_symm, W_gcn, W_dense):
    num_user = user_X.shape[0]
    x = jnp.concatenate([user_X, item_X], axis=0)
    n = x.shape[0]
    w = n + 8
    emb = _embed(x, W_gcn)
    lpad = _expand_rows_sc(P_symm, n, w)
    at, deg = _atilde(graph_A, mask_sub_adj, lpad)
    out = _spmm_head(at, emb, deg, W_dense)
    return out[:num_user], out[num_user:]
